# Initial kernel scaffold; baseline (speedup 1.0000x reference)
import jax
import jax.numpy as jnp
from jax.experimental import pallas as pl


def kernel(h, adj_hom, adj_het, t1_w, t1_b, gate_w_0, gate_b_0, gate_w_1, gate_b_1, t2_w, t2_b):
    raise NotImplementedError("write your pallas kernel here")


# speedup vs baseline: 53.0065x; 53.0065x over previous
import jax
import jax.numpy as jnp
from jax.experimental import pallas as pl
from jax.experimental.pallas import tpu as pltpu

EPS = 0.3


def _relu_linear_kernel(h_ref, w_ref, b_ref, o_ref):
    o_ref[...] = jax.nn.relu(
        jax.lax.dot_general(h_ref[...], w_ref[...], (((1,), (1,)), ((), ())),
                            preferred_element_type=jnp.float32) + b_ref[...])


def _degrees_kernel(ah_ref, at_ref, dh_ref, dt_ref):
    r = pl.program_id(0)

    @pl.when(r == 0)
    def _():
        dh_ref[...] = jnp.zeros_like(dh_ref)
        dt_ref[...] = jnp.zeros_like(dt_ref)

    dh_ref[...] += jnp.sum(ah_ref[...], axis=0, keepdims=True)
    dt_ref[...] += jnp.sum(at_ref[...], axis=0, keepdims=True)


def _gate_kernel(x_ref, gw_ref, gb_ref, ta_ref, tb_ref):
    gw = gw_ref[...]
    hid = x_ref.shape[1]
    gwa = gw[:, :hid]
    gwb = gw[:, hid:]
    x = x_ref[...]
    a = jnp.sum(x * gwa, axis=1, keepdims=True)
    b = jnp.sum(x * gwb, axis=1, keepdims=True)
    ta_ref[...] = jnp.tanh(a + gb_ref[0, 0])
    tb_ref[...] = jnp.tanh(b)


def _fa_kernel(ah_ref, at_ref, ta_ref, tb_ref, dhr_ref, dhc_ref, dtr_ref,
               dtc_ref, x_ref, raw_ref, o_ref):
    r = pl.program_id(0)
    nr = pl.num_programs(0)

    def nd(d):
        return jnp.where(d > 0, jax.lax.rsqrt(d), 0.0)

    ta = ta_ref[...]
    tb = tb_ref[...]
    t = (ta + tb) / (1.0 + ta * tb)
    wh = nd(dhr_ref[...]) * nd(dhc_ref[...])
    wt = nd(dtr_ref[...]) * nd(dtc_ref[...])
    w = (0.5 * t) * (ah_ref[...] * wh + at_ref[...] * wt)
    p = jax.lax.dot_general(w, x_ref[...], (((0,), (0,)), ((), ())),
                            preferred_element_type=jnp.float32)

    @pl.when(r == 0)
    def _():
        o_ref[...] = EPS * raw_ref[...]

    o_ref[...] += p


def _head_kernel(x_ref, w_ref, b_ref, o_ref):
    l = jax.lax.dot_general(x_ref[...], w_ref[...], (((1,), (1,)), ((), ())),
                            preferred_element_type=jnp.float32) + b_ref[...]
    m = jnp.max(l, axis=1, keepdims=True)
    o_ref[...] = l - m - jnp.log(jnp.sum(jnp.exp(l - m), axis=1, keepdims=True))


def kernel(h, adj_hom, adj_het, t1_w, t1_b, gate_w_0, gate_b_0, gate_w_1,
           gate_b_1, t2_w, t2_b):
    n, feat = h.shape
    hid = t1_w.shape[0]
    cls = t2_w.shape[0]
    f32 = jnp.float32

    blk = 1000 if n % 1000 == 0 else n
    nb = n // blk
    rblk = 80 if n % 80 == 0 else n
    nrb = n // rblk

    x0 = pl.pallas_call(
        _relu_linear_kernel,
        grid=(nb,),
        in_specs=[
            pl.BlockSpec((blk, feat), lambda i: (i, 0)),
            pl.BlockSpec((hid, feat), lambda i: (0, 0)),
            pl.BlockSpec((1, hid), lambda i: (0, 0)),
        ],
        out_specs=pl.BlockSpec((blk, hid), lambda i: (i, 0)),
        out_shape=jax.ShapeDtypeStruct((n, hid), f32),
    )(h, t1_w, t1_b.reshape(1, hid))

    dh, dt = pl.pallas_call(
        _degrees_kernel,
        grid=(nrb,),
        in_specs=[
            pl.BlockSpec((rblk, n), lambda r: (r, 0)),
            pl.BlockSpec((rblk, n), lambda r: (r, 0)),
        ],
        out_specs=[
            pl.BlockSpec((1, n), lambda r: (0, 0)),
            pl.BlockSpec((1, n), lambda r: (0, 0)),
        ],
        out_shape=[
            jax.ShapeDtypeStruct((1, n), f32),
            jax.ShapeDtypeStruct((1, n), f32),
        ],
        compiler_params=pltpu.CompilerParams(
            dimension_semantics=("arbitrary",)),
    )(adj_hom, adj_het)

    dh_c = dh
    dt_c = dt
    dh_r = dh.reshape(n, 1)
    dt_r = dt.reshape(n, 1)

    gate_fn = pl.pallas_call(
        _gate_kernel,
        grid=(nb,),
        in_specs=[
            pl.BlockSpec((blk, hid), lambda i: (i, 0)),
            pl.BlockSpec((1, 2 * hid), lambda i: (0, 0)),
            pl.BlockSpec((1, 1), lambda i: (0, 0)),
        ],
        out_specs=[
            pl.BlockSpec((blk, 1), lambda i: (i, 0)),
            pl.BlockSpec((blk, 1), lambda i: (i, 0)),
        ],
        out_shape=[
            jax.ShapeDtypeStruct((n, 1), f32),
            jax.ShapeDtypeStruct((n, 1), f32),
        ],
    )

    fa_fn = pl.pallas_call(
        _fa_kernel,
        grid=(nrb,),
        in_specs=[
            pl.BlockSpec((rblk, n), lambda r: (r, 0)),
            pl.BlockSpec((rblk, n), lambda r: (r, 0)),
            pl.BlockSpec((rblk, 1), lambda r: (r, 0)),
            pl.BlockSpec((1, n), lambda r: (0, 0)),
            pl.BlockSpec((rblk, 1), lambda r: (r, 0)),
            pl.BlockSpec((1, n), lambda r: (0, 0)),
            pl.BlockSpec((rblk, 1), lambda r: (r, 0)),
            pl.BlockSpec((1, n), lambda r: (0, 0)),
            pl.BlockSpec((rblk, hid), lambda r: (r, 0)),
            pl.BlockSpec((n, hid), lambda r: (0, 0)),
        ],
        out_specs=pl.BlockSpec((n, hid), lambda r: (0, 0)),
        out_shape=jax.ShapeDtypeStruct((n, hid), f32),
        compiler_params=pltpu.CompilerParams(
            dimension_semantics=("arbitrary",)),
    )

    x = x0
    for gw, gb in ((gate_w_0, gate_b_0), (gate_w_1, gate_b_1)):
        ta, tb = gate_fn(x, gw, gb.reshape(1, 1))
        x = fa_fn(adj_hom, adj_het, ta, tb.reshape(1, n), dh_r, dh_c, dt_r,
                  dt_c, x, x0)

    out = pl.pallas_call(
        _head_kernel,
        grid=(nb,),
        in_specs=[
            pl.BlockSpec((blk, hid), lambda i: (i, 0)),
            pl.BlockSpec((cls, hid), lambda i: (0, 0)),
            pl.BlockSpec((1, cls), lambda i: (0, 0)),
        ],
        out_specs=pl.BlockSpec((blk, cls), lambda i: (i, 0)),
        out_shape=jax.ShapeDtypeStruct((n, cls), f32),
    )(x, t2_w, t2_b.reshape(1, cls))

    return out
